# baseline (speedup 1.0000x reference)
import functools

import jax
import jax.numpy as jnp
from jax import lax
from jax.experimental import pallas as pl
from jax.experimental.pallas import tpu as pltpu
from jax.experimental.pallas import tpu_sc as plsc

_B = 1024
_SEQ = 50
_H = 128
_E = 64
_V = 100000
_VT = 2048
_NVT = (_V + _VT - 1) // _VT
_VPAD = _NVT * _VT

_BB = 256
_NBB = _B // _BB

_NC = 2
_NS = 16
_NW = _NC * _NS
_RPW = _B // _NW
_PADW = 64


def _emb_gather_body(table_hbm, idx_hbm, out_hbm, idx_v, rows_v, sem):
    wid = lax.axis_index("s") * _NC + lax.axis_index("c")
    base = wid * _RPW
    pltpu.sync_copy(idx_hbm.at[pl.ds(base, _RPW)], idx_v)
    pltpu.async_copy(table_hbm.at[idx_v], rows_v, sem).wait()
    pltpu.sync_copy(rows_v, out_hbm.at[pl.ds(base, _RPW)])


def _emb_gather(table, idx):
    mesh = plsc.VectorSubcoreMesh(core_axis_name="c", subcore_axis_name="s")
    return pl.kernel(
        _emb_gather_body,
        out_type=jax.ShapeDtypeStruct((_B, _E), jnp.float32),
        mesh=mesh,
        compiler_params=pltpu.CompilerParams(use_tc_tiling_on_sc=False),
        scratch_types=[
            pltpu.VMEM((_RPW,), jnp.int32),
            pltpu.VMEM((_RPW, _E), jnp.float32),
            pltpu.SemaphoreType.DMA,
        ],
    )(table, idx)


def _prep_body(emb_ref, w_ref, h_ref, wih_ref, whh_ref, bih_ref, bhh_ref,
               wc_ref, wcb_ref, enc_ref, eidx_ref, state_ref, scorec_ref):
    x = jnp.concatenate([emb_ref[...], w_ref[...]], axis=1)
    h = h_ref[...]
    gi = lax.dot_general(x, wih_ref[...], (((1,), (1,)), ((), ())),
                         preferred_element_type=jnp.float32) + bih_ref[...]
    gh = lax.dot_general(h, whh_ref[...], (((1,), (1,)), ((), ())),
                         preferred_element_type=jnp.float32) + bhh_ref[...]
    i_r = gi[:, :_H]
    i_z = gi[:, _H:2 * _H]
    i_n = gi[:, 2 * _H:]
    h_r = gh[:, :_H]
    h_z = gh[:, _H:2 * _H]
    h_n = gh[:, 2 * _H:]
    r = jax.nn.sigmoid(i_r + h_r)
    z = jax.nn.sigmoid(i_z + h_z)
    n = jnp.tanh(i_n + r * h_n)
    state = (1.0 - z) * n + z * h
    state_ref[...] = state

    enc = enc_ref[...]
    e2 = enc.reshape(_BB * _SEQ, _H)
    g = lax.dot_general(e2, wc_ref[...], (((1,), (1,)), ((), ())),
                        preferred_element_type=jnp.float32) + wcb_ref[...]
    sc3 = jnp.tanh(g).reshape(_BB, _SEQ, _H)
    cols = []
    for s in range(_SEQ):
        v = sc3[:, s, :]
        cols.append(jnp.sum(v * state, axis=1, keepdims=True))
    score_c = jnp.tanh(jnp.concatenate(cols, axis=1))
    mask = (eidx_ref[...] == 0).astype(jnp.float32) * (-1000.0)
    scorec_ref[...] = score_c + mask


def _prep(emb, w0, prev_state, W_ih, W_hh, b_ih, b_hh, Wc_w, Wc_b,
          encoded, encoded_idx):
    full = lambda shape: pl.BlockSpec(shape, lambda i: tuple(0 for _ in shape))
    return pl.pallas_call(
        _prep_body,
        grid=(_NBB,),
        in_specs=[
            pl.BlockSpec((_BB, _E), lambda i: (i, 0)),
            pl.BlockSpec((_BB, _H), lambda i: (i, 0)),
            pl.BlockSpec((_BB, _H), lambda i: (i, 0)),
            full((3 * _H, _E + _H)),
            full((3 * _H, _H)),
            full((1, 3 * _H)),
            full((1, 3 * _H)),
            full((_H, _H)),
            full((1, _H)),
            pl.BlockSpec((_BB, _SEQ, _H), lambda i: (i, 0, 0)),
            pl.BlockSpec((_BB, _SEQ), lambda i: (i, 0)),
        ],
        out_specs=[
            pl.BlockSpec((_BB, _H), lambda i: (i, 0)),
            pl.BlockSpec((_BB, _SEQ), lambda i: (i, 0)),
        ],
        out_shape=[
            jax.ShapeDtypeStruct((_B, _H), jnp.float32),
            jax.ShapeDtypeStruct((_B, _SEQ), jnp.float32),
        ],
    )(emb, w0, prev_state, W_ih, W_hh, b_ih, b_hh, Wc_w, Wc_b,
      encoded, encoded_idx)


def _sweep1_body(wo_ref, wob_ref, state_ref, scorec_ref,
                 m_out, invd_out, probc_out, m_ref, l_ref):
    i = pl.program_id(0)

    @pl.when(i == 0)
    def _():
        sc = scorec_ref[...]
        m0 = jnp.max(sc, axis=1, keepdims=True)
        l0 = jnp.sum(jnp.exp(sc - m0), axis=1, keepdims=True)
        m_ref[...] = m0
        l_ref[...] = l0

    s = lax.dot_general(state_ref[...], wo_ref[...], (((1,), (1,)), ((), ())),
                        preferred_element_type=jnp.float32) + wob_ref[...]
    cols = lax.broadcasted_iota(jnp.int32, (1, _VT), 1) + i * _VT
    s = jnp.where(cols < _V, s, -1e30)
    m_prev = m_ref[...]
    l_prev = l_ref[...]
    mt = jnp.max(s, axis=1, keepdims=True)
    m_new = jnp.maximum(m_prev, mt)
    l_new = (l_prev * jnp.exp(m_prev - m_new)
             + jnp.sum(jnp.exp(s - m_new), axis=1, keepdims=True))
    m_ref[...] = m_new
    l_ref[...] = l_new

    @pl.when(i == _NVT - 1)
    def _():
        M = m_ref[...]
        invd = 1.0 / l_ref[...]
        m_out[...] = M
        invd_out[...] = invd
        probc_out[...] = jnp.exp(scorec_ref[...] - M) * invd


def _sweep1(Wo_w, wob_pad, state, score_c):
    return pl.pallas_call(
        _sweep1_body,
        grid=(_NVT,),
        in_specs=[
            pl.BlockSpec((_VT, _H), lambda i: (i, 0)),
            pl.BlockSpec((1, _VT), lambda i: (0, i)),
            pl.BlockSpec((_B, _H), lambda i: (0, 0)),
            pl.BlockSpec((_B, _SEQ), lambda i: (0, 0)),
        ],
        out_specs=[
            pl.BlockSpec((_B, 1), lambda i: (0, 0)),
            pl.BlockSpec((_B, 1), lambda i: (0, 0)),
            pl.BlockSpec((_B, _SEQ), lambda i: (0, 0)),
        ],
        out_shape=[
            jax.ShapeDtypeStruct((_B, 1), jnp.float32),
            jax.ShapeDtypeStruct((_B, 1), jnp.float32),
            jax.ShapeDtypeStruct((_B, _SEQ), jnp.float32),
        ],
        scratch_shapes=[
            pltpu.VMEM((_B, 1), jnp.float32),
            pltpu.VMEM((_B, 1), jnp.float32),
        ],
    )(Wo_w, wob_pad, state, score_c)


def _sweep2_body(wo_ref, wob_ref, state_ref, m_ref, invd_ref, out_ref):
    s = lax.dot_general(state_ref[...], wo_ref[...], (((1,), (1,)), ((), ())),
                        preferred_element_type=jnp.float32) + wob_ref[...]
    out_ref[...] = jnp.exp(s - m_ref[...]) * invd_ref[...]


def _sweep2(Wo_w, wob_pad, state, mM, invd):
    return pl.pallas_call(
        _sweep2_body,
        grid=(_NVT,),
        in_specs=[
            pl.BlockSpec((_VT, _H), lambda i: (i, 0)),
            pl.BlockSpec((1, _VT), lambda i: (0, i)),
            pl.BlockSpec((_B, _H), lambda i: (0, 0)),
            pl.BlockSpec((_B, 1), lambda i: (0, 0)),
            pl.BlockSpec((_B, 1), lambda i: (0, 0)),
        ],
        out_specs=pl.BlockSpec((_B, _VT), lambda i: (0, i)),
        out_shape=jax.ShapeDtypeStruct((_B, _V), jnp.float32),
    )(Wo_w, wob_pad, state, mM, invd)


def _finalize_body(probc_ref, eidx_ref, iidx_ref, enc_ref,
                   wnew_ref, idxf_ref, valp_ref):
    pid = pl.program_id(0)
    pc = probc_ref[...]
    ei = eidx_ref[...]
    ii = iidx_ref[...]
    meq = (ei == ii).astype(jnp.float32)
    ssum = jnp.sum(meq, axis=1, keepdims=True)
    meq = jnp.where(ssum > 1.0, meq / jnp.maximum(ssum, 1e-9), meq)
    attn = pc * meq

    enc = enc_ref[...]
    wn = jnp.zeros((_BB, _H), jnp.float32)
    for s in range(_SEQ):
        wn = wn + attn[:, s:s + 1] * enc[:, s, :]
    wnew_ref[...] = wn

    combined = jnp.zeros((_BB, _SEQ), jnp.float32)
    for s in range(_SEQ):
        same = (ei == ei[:, s:s + 1]).astype(jnp.float32)
        combined = combined + pc[:, s:s + 1] * same
    idxp = jnp.concatenate([ei, ei[:, :_PADW - _SEQ]], axis=1)
    valp = jnp.concatenate([combined, combined[:, :_PADW - _SEQ]], axis=1)
    rowid = pid * _BB + lax.broadcasted_iota(jnp.int32, (_BB, 1), 0)
    idxf_ref[...] = idxp + rowid * _V
    valp_ref[...] = valp


def _finalize(probc, encoded_idx, iidx, encoded):
    return pl.pallas_call(
        _finalize_body,
        grid=(_NBB,),
        in_specs=[
            pl.BlockSpec((_BB, _SEQ), lambda i: (i, 0)),
            pl.BlockSpec((_BB, _SEQ), lambda i: (i, 0)),
            pl.BlockSpec((_BB, 1), lambda i: (i, 0)),
            pl.BlockSpec((_BB, _SEQ, _H), lambda i: (i, 0, 0)),
        ],
        out_specs=[
            pl.BlockSpec((_BB, _H), lambda i: (i, 0)),
            pl.BlockSpec((_BB, _PADW), lambda i: (i, 0)),
            pl.BlockSpec((_BB, _PADW), lambda i: (i, 0)),
        ],
        out_shape=[
            jax.ShapeDtypeStruct((_B, _H), jnp.float32),
            jax.ShapeDtypeStruct((_B, _PADW), jnp.int32),
            jax.ShapeDtypeStruct((_B, _PADW), jnp.float32),
        ],
    )(probc, encoded_idx, iidx, encoded)


def _sc_scatter(out_flat, idx_flat, vals):
    mesh = plsc.VectorSubcoreMesh(core_axis_name="c", subcore_axis_name="s")

    def stateful(refs):
        out_ref, idx_ref, val_ref = refs

        @pl.core_map(mesh)
        def _():
            def scoped(idx_v, val_v, gath_v, sem):
                wid = lax.axis_index("s") * _NC + lax.axis_index("c")
                base = wid * _RPW
                pltpu.sync_copy(idx_ref.at[pl.ds(base, _RPW)], idx_v)
                pltpu.sync_copy(val_ref.at[pl.ds(base, _RPW)], val_v)
                descs = []
                for r in range(_RPW):
                    descs.append(pltpu.async_copy(
                        out_ref.at[idx_v.at[r]], gath_v.at[r], sem))
                for d in descs:
                    d.wait()
                for r in range(_RPW):
                    for j in range(_PADW // 16):
                        sl = pl.ds(j * 16, 16)
                        gath_v[r, sl] = gath_v[r, sl] + val_v[r, sl]
                descs = []
                for r in range(_RPW):
                    descs.append(pltpu.async_copy(
                        gath_v.at[r], out_ref.at[idx_v.at[r]], sem))
                for d in descs:
                    d.wait()

            pl.run_scoped(
                scoped,
                pltpu.VMEM((_RPW, _PADW), jnp.int32),
                pltpu.VMEM((_RPW, _PADW), jnp.float32),
                pltpu.VMEM((_RPW, _PADW), jnp.float32),
                pltpu.SemaphoreType.DMA,
            )

    out2, _, _ = pl.run_state(stateful)((out_flat, idx_flat, vals))
    return out2


def kernel(input_idx, encoded, encoded_idx, prev_state, weighted, order,
           embed_table, W_ih, W_hh, b_ih, b_hh, Wo_w, Wo_b, Wc_w, Wc_b):
    w0 = jnp.where(order == 0, 0.0, weighted[:, 0, :])
    idx_i = input_idx.astype(jnp.int32)
    eidx = encoded_idx.astype(jnp.int32)

    emb = _emb_gather(embed_table, idx_i)
    state, score_c = _prep(
        emb, w0, prev_state, W_ih, W_hh,
        b_ih.reshape(1, -1), b_hh.reshape(1, -1),
        Wc_w, Wc_b.reshape(1, -1), encoded, eidx)

    wob_pad = jnp.pad(Wo_b, (0, _VPAD - _V)).reshape(1, _VPAD)
    mM, invd, probc = _sweep1(Wo_w, wob_pad, state, score_c)
    out2d = _sweep2(Wo_w, wob_pad, state, mM, invd)
    wnew, idxf, valp = _finalize(probc, eidx, idx_i.reshape(_B, 1), encoded)

    out_flat = _sc_scatter(out2d.reshape(_B * _V), idxf, valp)
    return out_flat.reshape(_B, 1, _V), state, wnew[:, None, :]


# speedup vs baseline: 1.0000x; 1.0000x over previous
import functools

import jax
import jax.numpy as jnp
from jax import lax
from jax.experimental import pallas as pl
from jax.experimental.pallas import tpu as pltpu
from jax.experimental.pallas import tpu_sc as plsc

_B = 1024
_SEQ = 50
_H = 128
_E = 64
_V = 100000
_VT = 2048
_NVT = (_V + _VT - 1) // _VT
_VPAD = _NVT * _VT

_BB = 256
_NBB = _B // _BB

_NC = 2
_NS = 16
_NW = _NC * _NS
_RPW = _B // _NW
_PADW = 64


def _emb_gather_body(table_hbm, idx_hbm, out_hbm, idx_v, rows_v, sem):
    wid = lax.axis_index("s") * _NC + lax.axis_index("c")
    base = wid * _RPW
    pltpu.sync_copy(idx_hbm.at[pl.ds(base, _RPW)], idx_v)
    pltpu.async_copy(table_hbm.at[idx_v], rows_v, sem).wait()
    pltpu.sync_copy(rows_v, out_hbm.at[pl.ds(base, _RPW)])


def _emb_gather(table, idx):
    mesh = plsc.VectorSubcoreMesh(core_axis_name="c", subcore_axis_name="s")
    return pl.kernel(
        _emb_gather_body,
        out_type=jax.ShapeDtypeStruct((_B, _E), jnp.float32),
        mesh=mesh,
        compiler_params=pltpu.CompilerParams(use_tc_tiling_on_sc=False),
        scratch_types=[
            pltpu.VMEM((_RPW,), jnp.int32),
            pltpu.VMEM((_RPW, _E), jnp.float32),
            pltpu.SemaphoreType.DMA,
        ],
    )(table, idx)


def _prep_body(emb_ref, w_ref, h_ref, wih_ref, whh_ref, bih_ref, bhh_ref,
               wc_ref, wcb_ref, enc_ref, eidx_ref, state_ref, scorec_ref):
    x = jnp.concatenate([emb_ref[...], w_ref[...]], axis=1)
    h = h_ref[...]
    gi = lax.dot_general(x, wih_ref[...], (((1,), (1,)), ((), ())),
                         preferred_element_type=jnp.float32) + bih_ref[...]
    gh = lax.dot_general(h, whh_ref[...], (((1,), (1,)), ((), ())),
                         preferred_element_type=jnp.float32) + bhh_ref[...]
    i_r = gi[:, :_H]
    i_z = gi[:, _H:2 * _H]
    i_n = gi[:, 2 * _H:]
    h_r = gh[:, :_H]
    h_z = gh[:, _H:2 * _H]
    h_n = gh[:, 2 * _H:]
    r = jax.nn.sigmoid(i_r + h_r)
    z = jax.nn.sigmoid(i_z + h_z)
    n = jnp.tanh(i_n + r * h_n)
    state = (1.0 - z) * n + z * h
    state_ref[...] = state

    enc = enc_ref[...]
    e2 = enc.reshape(_BB * _SEQ, _H)
    g = lax.dot_general(e2, wc_ref[...], (((1,), (1,)), ((), ())),
                        preferred_element_type=jnp.float32) + wcb_ref[...]
    sc3 = jnp.tanh(g).reshape(_BB, _SEQ, _H)
    cols = []
    for s in range(_SEQ):
        v = sc3[:, s, :]
        cols.append(jnp.sum(v * state, axis=1, keepdims=True))
    score_c = jnp.tanh(jnp.concatenate(cols, axis=1))
    mask = (eidx_ref[...] == 0).astype(jnp.float32) * (-1000.0)
    scorec_ref[...] = score_c + mask


def _prep(emb, w0, prev_state, W_ih, W_hh, b_ih, b_hh, Wc_w, Wc_b,
          encoded, encoded_idx):
    full = lambda shape: pl.BlockSpec(shape, lambda i: tuple(0 for _ in shape))
    return pl.pallas_call(
        _prep_body,
        grid=(_NBB,),
        in_specs=[
            pl.BlockSpec((_BB, _E), lambda i: (i, 0)),
            pl.BlockSpec((_BB, _H), lambda i: (i, 0)),
            pl.BlockSpec((_BB, _H), lambda i: (i, 0)),
            full((3 * _H, _E + _H)),
            full((3 * _H, _H)),
            full((1, 3 * _H)),
            full((1, 3 * _H)),
            full((_H, _H)),
            full((1, _H)),
            pl.BlockSpec((_BB, _SEQ, _H), lambda i: (i, 0, 0)),
            pl.BlockSpec((_BB, _SEQ), lambda i: (i, 0)),
        ],
        out_specs=[
            pl.BlockSpec((_BB, _H), lambda i: (i, 0)),
            pl.BlockSpec((_BB, _SEQ), lambda i: (i, 0)),
        ],
        out_shape=[
            jax.ShapeDtypeStruct((_B, _H), jnp.float32),
            jax.ShapeDtypeStruct((_B, _SEQ), jnp.float32),
        ],
    )(emb, w0, prev_state, W_ih, W_hh, b_ih, b_hh, Wc_w, Wc_b,
      encoded, encoded_idx)


def _sweep1_body(wo_ref, wob_ref, state_ref, scorec_ref,
                 m_out, invd_out, probc_out, m_ref, l_ref):
    i = pl.program_id(0)

    @pl.when(i == 0)
    def _():
        sc = scorec_ref[...]
        m0 = jnp.max(sc, axis=1, keepdims=True)
        l0 = jnp.sum(jnp.exp(sc - m0), axis=1, keepdims=True)
        m_ref[...] = m0
        l_ref[...] = l0

    s = lax.dot_general(state_ref[...], wo_ref[...], (((1,), (1,)), ((), ())),
                        preferred_element_type=jnp.float32) + wob_ref[...]
    cols = lax.broadcasted_iota(jnp.int32, (1, _VT), 1) + i * _VT
    s = jnp.where(cols < _V, s, -1e30)
    m_prev = m_ref[...]
    l_prev = l_ref[...]
    mt = jnp.max(s, axis=1, keepdims=True)
    m_new = jnp.maximum(m_prev, mt)
    l_new = (l_prev * jnp.exp(m_prev - m_new)
             + jnp.sum(jnp.exp(s - m_new), axis=1, keepdims=True))
    m_ref[...] = m_new
    l_ref[...] = l_new

    @pl.when(i == _NVT - 1)
    def _():
        M = m_ref[...]
        invd = 1.0 / l_ref[...]
        m_out[...] = M
        invd_out[...] = invd
        probc_out[...] = jnp.exp(scorec_ref[...] - M) * invd


def _sweep1(Wo_w, wob_pad, state, score_c):
    return pl.pallas_call(
        _sweep1_body,
        grid=(_NVT,),
        in_specs=[
            pl.BlockSpec((_VT, _H), lambda i: (i, 0)),
            pl.BlockSpec((1, _VT), lambda i: (0, i)),
            pl.BlockSpec((_B, _H), lambda i: (0, 0)),
            pl.BlockSpec((_B, _SEQ), lambda i: (0, 0)),
        ],
        out_specs=[
            pl.BlockSpec((_B, 1), lambda i: (0, 0)),
            pl.BlockSpec((_B, 1), lambda i: (0, 0)),
            pl.BlockSpec((_B, _SEQ), lambda i: (0, 0)),
        ],
        out_shape=[
            jax.ShapeDtypeStruct((_B, 1), jnp.float32),
            jax.ShapeDtypeStruct((_B, 1), jnp.float32),
            jax.ShapeDtypeStruct((_B, _SEQ), jnp.float32),
        ],
        scratch_shapes=[
            pltpu.VMEM((_B, 1), jnp.float32),
            pltpu.VMEM((_B, 1), jnp.float32),
        ],
    )(Wo_w, wob_pad, state, score_c)


def _sweep2_body(wo_ref, wob_ref, state_ref, m_ref, invd_ref, out_ref):
    s = lax.dot_general(state_ref[...], wo_ref[...], (((1,), (1,)), ((), ())),
                        preferred_element_type=jnp.float32) + wob_ref[...]
    out_ref[...] = jnp.exp(s - m_ref[...]) * invd_ref[...]


def _sweep2(Wo_w, wob_pad, state, mM, invd):
    return pl.pallas_call(
        _sweep2_body,
        grid=(_NVT,),
        in_specs=[
            pl.BlockSpec((_VT, _H), lambda i: (i, 0)),
            pl.BlockSpec((1, _VT), lambda i: (0, i)),
            pl.BlockSpec((_B, _H), lambda i: (0, 0)),
            pl.BlockSpec((_B, 1), lambda i: (0, 0)),
            pl.BlockSpec((_B, 1), lambda i: (0, 0)),
        ],
        out_specs=pl.BlockSpec((_B, _VT), lambda i: (0, i)),
        out_shape=jax.ShapeDtypeStruct((_B, _V), jnp.float32),
    )(Wo_w, wob_pad, state, mM, invd)


def _finalize_body(probc_ref, eidx_ref, iidx_ref, enc_ref,
                   wnew_ref, idxf_ref, valp_ref):
    pid = pl.program_id(0)
    pc = probc_ref[...]
    ei = eidx_ref[...]
    ii = iidx_ref[...]
    meq = (ei == ii).astype(jnp.float32)
    ssum = jnp.sum(meq, axis=1, keepdims=True)
    meq = jnp.where(ssum > 1.0, meq / jnp.maximum(ssum, 1e-9), meq)
    attn = pc * meq

    enc = enc_ref[...]
    wn = jnp.zeros((_BB, _H), jnp.float32)
    for s in range(_SEQ):
        wn = wn + attn[:, s:s + 1] * enc[:, s, :]
    wnew_ref[...] = wn

    combined = jnp.zeros((_BB, _SEQ), jnp.float32)
    for s in range(_SEQ):
        same = (ei == ei[:, s:s + 1]).astype(jnp.float32)
        combined = combined + pc[:, s:s + 1] * same
    idxp = jnp.concatenate([ei, ei[:, :_PADW - _SEQ]], axis=1)
    valp = jnp.concatenate([combined, combined[:, :_PADW - _SEQ]], axis=1)
    idxf_ref[...] = idxp
    valp_ref[...] = valp


def _finalize(probc, encoded_idx, iidx, encoded):
    return pl.pallas_call(
        _finalize_body,
        grid=(_NBB,),
        in_specs=[
            pl.BlockSpec((_BB, _SEQ), lambda i: (i, 0)),
            pl.BlockSpec((_BB, _SEQ), lambda i: (i, 0)),
            pl.BlockSpec((_BB, 1), lambda i: (i, 0)),
            pl.BlockSpec((_BB, _SEQ, _H), lambda i: (i, 0, 0)),
        ],
        out_specs=[
            pl.BlockSpec((_BB, _H), lambda i: (i, 0)),
            pl.BlockSpec((_BB, _PADW), lambda i: (i, 0)),
            pl.BlockSpec((_BB, _PADW), lambda i: (i, 0)),
        ],
        out_shape=[
            jax.ShapeDtypeStruct((_B, _H), jnp.float32),
            jax.ShapeDtypeStruct((_B, _PADW), jnp.int32),
            jax.ShapeDtypeStruct((_B, _PADW), jnp.float32),
        ],
    )(probc, encoded_idx, iidx, encoded)


def _sc_scatter(out2d, idx_flat, vals):
    mesh = plsc.VectorSubcoreMesh(core_axis_name="c", subcore_axis_name="s")

    def stateful(refs):
        out_ref, idx_ref, val_ref = refs

        @pl.core_map(
            mesh,
            compiler_params=pltpu.CompilerParams(use_tc_tiling_on_sc=False),
        )
        def _():
            def scoped(idx_row, val_row, gath_row, sem):
                wid = lax.axis_index("s") * _NC + lax.axis_index("c")
                base = wid * _RPW
                for r in range(_RPW):
                    pltpu.sync_copy(idx_ref.at[base + r], idx_row)
                    pltpu.sync_copy(val_ref.at[base + r], val_row)
                    pltpu.async_copy(
                        out_ref.at[base + r].at[idx_row], gath_row, sem
                    ).wait()
                    for j in range(_PADW // 16):
                        sl = pl.ds(j * 16, 16)
                        gath_row[sl] = gath_row[sl] + val_row[sl]
                    pltpu.async_copy(
                        gath_row, out_ref.at[base + r].at[idx_row], sem
                    ).wait()

            pl.run_scoped(
                scoped,
                pltpu.VMEM((_PADW,), jnp.int32),
                pltpu.VMEM((_PADW,), jnp.float32),
                pltpu.VMEM((_PADW,), jnp.float32),
                pltpu.SemaphoreType.DMA,
            )

    out2, _, _ = pl.run_state(stateful)((out2d, idx_flat, vals))
    return out2


def kernel(input_idx, encoded, encoded_idx, prev_state, weighted, order,
           embed_table, W_ih, W_hh, b_ih, b_hh, Wo_w, Wo_b, Wc_w, Wc_b):
    w0 = jnp.where(order == 0, 0.0, weighted[:, 0, :])
    idx_i = input_idx.astype(jnp.int32)
    eidx = encoded_idx.astype(jnp.int32)

    emb = _emb_gather(embed_table, idx_i)
    state, score_c = _prep(
        emb, w0, prev_state, W_ih, W_hh,
        b_ih.reshape(1, -1), b_hh.reshape(1, -1),
        Wc_w, Wc_b.reshape(1, -1), encoded, eidx)

    wob_pad = jnp.pad(Wo_b, (0, _VPAD - _V)).reshape(1, _VPAD)
    mM, invd, probc = _sweep1(Wo_w, wob_pad, state, score_c)
    out2d = _sweep2(Wo_w, wob_pad, state, mM, invd)
    wnew, idxf, valp = _finalize(probc, eidx, idx_i.reshape(_B, 1), encoded)

    out_fin = _sc_scatter(out2d, idxf, valp)
    return out_fin[:, None, :], state, wnew[:, None, :]
